# baseline (speedup 1.0000x reference)
import functools

import jax
import jax.numpy as jnp
from jax import lax
from jax.experimental import pallas as pl
from jax.experimental.pallas import tpu as pltpu
from jax.experimental.pallas import tpu_sc as plsc

_N = 10000
_E = 320000
_NC = 2
_NS = 16
_B = 64
_NP = 10112
_RPS = _NP // _NS
_EP = 327680
_EBLK = _EP // _B
_DH = 128
_K = 4


def _make_deg_kernel():
    mesh = plsc.VectorSubcoreMesh(core_axis_name="c", subcore_axis_name="s")
    nblk = _EBLK // (_NC * _NS)

    @functools.partial(
        pl.kernel,
        out_type=jax.ShapeDtypeStruct((_NC * _NP, 16), jnp.float32),
        mesh=mesh,
        scratch_types=[
            pltpu.VMEM((nblk, _B), jnp.int32),
            pltpu.VMEM((_B, 16), jnp.float32),
            pltpu.VMEM_SHARED((_NP, 16), jnp.float32),
        ],
        compiler_params=pltpu.CompilerParams(use_tc_tiling_on_sc=False),
    )
    def deg_kernel(dst_hbm, ones_hbm, zeros_hbm, out_hbm, dstv, onesv, acc):
        c = lax.axis_index("c")
        s = lax.axis_index("s")
        w = c * _NS + s
        pltpu.sync_copy(dst_hbm.at[pl.ds(w * nblk, nblk)], dstv)
        pltpu.sync_copy(ones_hbm, onesv)
        pltpu.sync_copy(zeros_hbm.at[pl.ds(s * _RPS, _RPS)],
                        acc.at[pl.ds(s * _RPS, _RPS)])
        plsc.subcore_barrier()

        @pl.loop(0, nblk)
        def _(j):
            pltpu.sync_copy(onesv, acc.at[dstv.at[j]], add=True)

        plsc.subcore_barrier()
        pltpu.sync_copy(acc.at[pl.ds(s * _RPS, _RPS)],
                        out_hbm.at[pl.ds(c * _NP + s * _RPS, _RPS)])

    return deg_kernel


def _make_agg_kernel():
    mesh = plsc.VectorSubcoreMesh(core_axis_name="c", subcore_axis_name="s")
    nblk = _EBLK // _NS

    @functools.partial(
        pl.kernel,
        out_type=jax.ShapeDtypeStruct((_NC * _NP, _DH), jnp.float32),
        mesh=mesh,
        scratch_types=[
            pltpu.VMEM((_K, 2, _B), jnp.int32),
            pltpu.VMEM((_K, _B, _DH), jnp.float32),
            pltpu.SemaphoreType.DMA((_K,)),
            pltpu.SemaphoreType.DMA((_K,)),
            pltpu.SemaphoreType.DMA((_K,)),
            pltpu.VMEM_SHARED((_NP, _DH), jnp.float32),
        ],
        compiler_params=pltpu.CompilerParams(use_tc_tiling_on_sc=False),
    )
    def agg_kernel(h_hbm, idx_hbm, out_hbm, idxv, rows, isem, gsem, ssem, acc):
        c = lax.axis_index("c")
        s = lax.axis_index("s")
        base = c * _EBLK + s * nblk
        pltpu.sync_copy(h_hbm.at[pl.ds(c * _NP + s * _RPS, _RPS)],
                        acc.at[pl.ds(s * _RPS, _RPS)])
        plsc.subcore_barrier()

        for k in range(_K):
            pltpu.async_copy(idx_hbm.at[base + k], idxv.at[k], isem.at[k])

        @pl.loop(0, nblk, step=_K)
        def _(j):
            for k in range(_K):
                pltpu.make_async_copy(idx_hbm.at[base], idxv.at[k],
                                      isem.at[k]).wait()
                pltpu.async_copy(h_hbm.at[idxv.at[k].at[0]], rows.at[k],
                                 gsem.at[k])
            for k in range(_K):
                pltpu.make_async_copy(h_hbm.at[idxv.at[k].at[0]], rows.at[k],
                                      gsem.at[k]).wait()
                pltpu.async_copy(rows.at[k], acc.at[idxv.at[k].at[1]],
                                 ssem.at[k], add=True)
            for k in range(_K):
                pltpu.make_async_copy(rows.at[k], acc.at[idxv.at[k].at[1]],
                                      ssem.at[k]).wait()

                @pl.when(j + _K < nblk)
                def _():
                    pltpu.async_copy(idx_hbm.at[base + j + _K + k],
                                     idxv.at[k], isem.at[k])

        plsc.subcore_barrier()
        pltpu.sync_copy(acc.at[pl.ds(s * _RPS, _RPS)],
                        out_hbm.at[pl.ds(c * _NP + s * _RPS, _RPS)])

    return agg_kernel


_deg = _make_deg_kernel()
_agg = _make_agg_kernel()


def _mm_body(x_ref, w_ref, o_ref):
    o_ref[...] = jnp.dot(x_ref[...], w_ref[...],
                         preferred_element_type=jnp.float32)


def _mm(x, w):
    return pl.pallas_call(
        _mm_body,
        out_shape=jax.ShapeDtypeStruct((x.shape[0], w.shape[1]), jnp.float32),
    )(x, w)


def _scale_body(h_ref, cnt_ref, hcat_ref, d_ref):
    c0 = cnt_ref[0:_N, 0:1]
    c1 = cnt_ref[_NP:_NP + _N, 0:1]
    d = lax.rsqrt(1.0 + c0 + c1)
    d_ref[...] = d
    hs = h_ref[...] * d
    hcat_ref[0:_N, :] = hs[:, 0:128]
    hcat_ref[_NP:_NP + _N, :] = hs[:, 128:256]


def _scale(h, cnt):
    return pl.pallas_call(
        _scale_body,
        out_shape=(jax.ShapeDtypeStruct((2 * _NP, _DH), jnp.float32),
                   jax.ShapeDtypeStruct((_N, 1), jnp.float32)),
    )(h, cnt)


def _mm2_body(a_ref, d_ref, b1_ref, w2_ref, o_ref):
    d = d_ref[...]
    h1lo = jnp.maximum(a_ref[0:_N, :] * d + b1_ref[0, 0:128], 0.0)
    h1hi = jnp.maximum(a_ref[_NP:_NP + _N, :] * d + b1_ref[0, 128:256], 0.0)
    h2 = (jnp.dot(h1lo, w2_ref[0:128, :], preferred_element_type=jnp.float32)
          + jnp.dot(h1hi, w2_ref[128:256, :], preferred_element_type=jnp.float32))
    h2 = h2 * d
    o_ref[0:_N, :] = h2
    o_ref[_NP:_NP + _N, :] = h2


def _mm2(a1, d, b1, w2):
    return pl.pallas_call(
        _mm2_body,
        out_shape=jax.ShapeDtypeStruct((2 * _NP, _DH), jnp.float32),
    )(a1, d, b1, w2)


def _fin_body(a_ref, d_ref, b2_ref, o_ref):
    d = d_ref[...]
    o_ref[...] = a_ref[0:_N, :] * d + b2_ref[...]


def _fin(acc2, d, b2):
    return pl.pallas_call(
        _fin_body,
        out_shape=jax.ShapeDtypeStruct((_N, 128), jnp.float32),
    )(acc2, d, b2)


def kernel(x, edge_index, W1, b1, W2, b2):
    src = edge_index[0]
    dst = edge_index[1]
    npad = _EP - _E
    srcp = jnp.concatenate([src, jnp.zeros((npad,), jnp.int32)])
    dstp = jnp.concatenate([dst, jnp.full((npad,), _N, jnp.int32)])
    dst2 = dstp.reshape(_EBLK, _B)
    pack = jnp.stack([jnp.stack([srcp, dstp]),
                      jnp.stack([srcp + _NP, dstp])])
    idx2 = (pack.reshape(2, 2, _EBLK, _B)
            .transpose(0, 2, 1, 3).reshape(2 * _EBLK, 2, _B))
    ones16 = jnp.ones((_B, 16), jnp.float32)
    zeros16 = jnp.zeros((_NP, 16), jnp.float32)

    cnt = _deg(dst2, ones16, zeros16)
    h = _mm(x, W1)
    hcat, d = _scale(h, cnt)
    a1 = _agg(hcat, idx2)
    h2cat = _mm2(a1, d, b1.reshape(1, -1), W2)
    acc2 = _agg(h2cat, idx2)
    return _fin(acc2, d, b2.reshape(1, -1))


# speedup vs baseline: 1.9060x; 1.9060x over previous
import functools

import jax
import jax.numpy as jnp
from jax import lax
from jax.experimental import pallas as pl
from jax.experimental.pallas import tpu as pltpu
from jax.experimental.pallas import tpu_sc as plsc

_N = 10000
_E = 320000
_NC = 2
_NS = 16
_B = 128
_NP = 10112
_RPS = _NP // _NS
_EP = 327680
_EBLK = _EP // _B
_DH = 64
_K = 4


def _make_deg_kernel():
    mesh = plsc.VectorSubcoreMesh(core_axis_name="c", subcore_axis_name="s")
    nblk = _EBLK // (_NC * _NS)

    @functools.partial(
        pl.kernel,
        out_type=jax.ShapeDtypeStruct((_NC * _NP, 16), jnp.float32),
        mesh=mesh,
        scratch_types=[
            pltpu.VMEM((nblk, _B), jnp.int32),
            pltpu.VMEM((_B, 16), jnp.float32),
            pltpu.VMEM_SHARED((_NP, 16), jnp.float32),
        ],
        compiler_params=pltpu.CompilerParams(use_tc_tiling_on_sc=False),
    )
    def deg_kernel(dst_hbm, ones_hbm, zeros_hbm, out_hbm, dstv, onesv, acc):
        c = lax.axis_index("c")
        s = lax.axis_index("s")
        w = c * _NS + s
        pltpu.sync_copy(dst_hbm.at[pl.ds(w * nblk, nblk)], dstv)
        pltpu.sync_copy(ones_hbm, onesv)
        pltpu.sync_copy(zeros_hbm.at[pl.ds(s * _RPS, _RPS)],
                        acc.at[pl.ds(s * _RPS, _RPS)])
        plsc.subcore_barrier()

        @pl.loop(0, nblk)
        def _(j):
            pltpu.sync_copy(onesv, acc.at[dstv.at[j]], add=True)

        plsc.subcore_barrier()
        pltpu.sync_copy(acc.at[pl.ds(s * _RPS, _RPS)],
                        out_hbm.at[pl.ds(c * _NP + s * _RPS, _RPS)])

    return deg_kernel


def _make_agg_kernel():
    mesh = plsc.VectorSubcoreMesh(core_axis_name="c", subcore_axis_name="s")
    nblk = _EBLK // _NS

    @functools.partial(
        pl.kernel,
        out_type=jax.ShapeDtypeStruct((_NC * _NP, _DH), jnp.float32),
        mesh=mesh,
        scratch_types=[
            pltpu.VMEM((nblk, _B), jnp.int32),
            pltpu.VMEM((nblk, _B), jnp.int32),
            pltpu.VMEM((_K, _B, _DH), jnp.float32),
            pltpu.SemaphoreType.DMA((_K,)),
            pltpu.SemaphoreType.DMA((_K,)),
            pltpu.VMEM_SHARED((_NP, _DH), jnp.float32),
        ],
        compiler_params=pltpu.CompilerParams(use_tc_tiling_on_sc=False),
    )
    def agg_kernel(v_hbm, src_hbm, dst_hbm, out_hbm, srcv, dstv, rows,
                   gsem, ssem, acc):
        c = lax.axis_index("c")
        s = lax.axis_index("s")
        pltpu.sync_copy(src_hbm.at[pl.ds(c * _EBLK + s * nblk, nblk)], srcv)
        pltpu.sync_copy(dst_hbm.at[pl.ds(s * nblk, nblk)], dstv)
        pltpu.sync_copy(v_hbm.at[pl.ds(c * _NP + s * _RPS, _RPS)],
                        acc.at[pl.ds(s * _RPS, _RPS)])
        plsc.subcore_barrier()

        for b in range(_K):
            pltpu.async_copy(v_hbm.at[srcv.at[b]], rows.at[b], gsem.at[b])

        @pl.loop(0, nblk, step=_K)
        def _(j):
            for b in range(_K):
                pltpu.make_async_copy(v_hbm.at[srcv.at[0]], rows.at[b],
                                      gsem.at[b]).wait()
                pltpu.async_copy(rows.at[b], acc.at[dstv.at[j + b]],
                                 ssem.at[b], add=True)
            for b in range(_K):
                pltpu.make_async_copy(rows.at[b], acc.at[dstv.at[0]],
                                      ssem.at[b]).wait()

                @pl.when(j + _K < nblk)
                def _():
                    pltpu.async_copy(v_hbm.at[srcv.at[j + _K + b]],
                                     rows.at[b], gsem.at[b])

        plsc.subcore_barrier()
        pltpu.sync_copy(acc.at[pl.ds(s * _RPS, _RPS)],
                        out_hbm.at[pl.ds(c * _NP + s * _RPS, _RPS)])

    return agg_kernel


_deg = _make_deg_kernel()
_agg = _make_agg_kernel()


def _scale0_body(x_ref, cnt_ref, xs_ref, d_ref):
    c0 = cnt_ref[0:_N, 0:1]
    c1 = cnt_ref[_NP:_NP + _N, 0:1]
    d = lax.rsqrt(1.0 + c0 + c1)
    d_ref[...] = d
    xs = x_ref[...] * d
    xs_ref[0:_N, :] = xs[:, 0:64]
    xs_ref[_NP:_NP + _N, :] = xs[:, 64:128]


def _scale0(x, cnt):
    return pl.pallas_call(
        _scale0_body,
        out_shape=(jax.ShapeDtypeStruct((2 * _NP, _DH), jnp.float32),
                   jax.ShapeDtypeStruct((_N, 1), jnp.float32)),
    )(x, cnt)


def _mid_body(a_ref, d_ref, b1_ref, w1_ref, w2_ref, o_ref):
    d = d_ref[...]
    aggx = jnp.concatenate([a_ref[0:_N, :], a_ref[_NP:_NP + _N, :]], axis=1)
    h1 = jnp.maximum(
        jnp.dot(aggx * d, w1_ref[...], preferred_element_type=jnp.float32)
        + b1_ref[...], 0.0)
    g = jnp.dot(h1, w2_ref[...], preferred_element_type=jnp.float32) * d
    o_ref[0:_N, :] = g[:, 0:64]
    o_ref[_NP:_NP + _N, :] = g[:, 64:128]


def _mid(a1, d, b1, w1, w2):
    return pl.pallas_call(
        _mid_body,
        out_shape=jax.ShapeDtypeStruct((2 * _NP, _DH), jnp.float32),
    )(a1, d, b1, w1, w2)


def _fin_body(a_ref, d_ref, b2_ref, o_ref):
    d = d_ref[...]
    lo = a_ref[0:_N, :]
    hi = a_ref[_NP:_NP + _N, :]
    o_ref[...] = jnp.concatenate([lo, hi], axis=1) * d + b2_ref[...]


def _fin(acc2, d, b2):
    return pl.pallas_call(
        _fin_body,
        out_shape=jax.ShapeDtypeStruct((_N, 128), jnp.float32),
    )(acc2, d, b2)


def kernel(x, edge_index, W1, b1, W2, b2):
    src = edge_index[0]
    dst = edge_index[1]
    npad = _EP - _E
    srcp = jnp.concatenate([src, jnp.zeros((npad,), jnp.int32)])
    dstp = jnp.concatenate([dst, jnp.full((npad,), _N, jnp.int32)])
    dst2 = dstp.reshape(_EBLK, _B)
    src2 = jnp.concatenate([srcp, srcp + _NP]).reshape(2 * _EBLK, _B)
    ones16 = jnp.ones((_B, 16), jnp.float32)
    zeros16 = jnp.zeros((_NP, 16), jnp.float32)

    cnt = _deg(dst2, ones16, zeros16)
    xs, d = _scale0(x, cnt)
    a1 = _agg(xs, src2, dst2)
    g = _mid(a1, d, b1.reshape(1, -1), W1, W2)
    a2 = _agg(g, src2, dst2)
    return _fin(a2, d, b2.reshape(1, -1))


# speedup vs baseline: 1.9155x; 1.0050x over previous
import functools

import jax
import jax.numpy as jnp
from jax import lax
from jax.experimental import pallas as pl
from jax.experimental.pallas import tpu as pltpu
from jax.experimental.pallas import tpu_sc as plsc

_N = 10000
_E = 320000
_NC = 2
_NS = 16
_B = 128
_NP = 10112
_RPS = _NP // _NS
_EP = 327680
_EBLK = _EP // _B
_DH = 64
_K = 5


def _make_deg_kernel():
    mesh = plsc.VectorSubcoreMesh(core_axis_name="c", subcore_axis_name="s")
    nblk = _EBLK // (_NC * _NS)

    @functools.partial(
        pl.kernel,
        out_type=jax.ShapeDtypeStruct((_NC * _NP, 16), jnp.float32),
        mesh=mesh,
        scratch_types=[
            pltpu.VMEM((nblk, _B), jnp.int32),
            pltpu.VMEM((_B, 16), jnp.float32),
            pltpu.VMEM_SHARED((_NP, 16), jnp.float32),
        ],
        compiler_params=pltpu.CompilerParams(use_tc_tiling_on_sc=False),
    )
    def deg_kernel(dst_hbm, ones_hbm, zeros_hbm, out_hbm, dstv, onesv, acc):
        c = lax.axis_index("c")
        s = lax.axis_index("s")
        w = c * _NS + s
        pltpu.sync_copy(dst_hbm.at[pl.ds(w * nblk, nblk)], dstv)
        pltpu.sync_copy(ones_hbm, onesv)
        pltpu.sync_copy(zeros_hbm.at[pl.ds(s * _RPS, _RPS)],
                        acc.at[pl.ds(s * _RPS, _RPS)])
        plsc.subcore_barrier()

        @pl.loop(0, nblk)
        def _(j):
            pltpu.sync_copy(onesv, acc.at[dstv.at[j]], add=True)

        plsc.subcore_barrier()
        pltpu.sync_copy(acc.at[pl.ds(s * _RPS, _RPS)],
                        out_hbm.at[pl.ds(c * _NP + s * _RPS, _RPS)])

    return deg_kernel


def _make_agg_kernel():
    mesh = plsc.VectorSubcoreMesh(core_axis_name="c", subcore_axis_name="s")
    nblk = _EBLK // _NS

    @functools.partial(
        pl.kernel,
        out_type=jax.ShapeDtypeStruct((_NC * _NP, _DH), jnp.float32),
        mesh=mesh,
        scratch_types=[
            pltpu.VMEM((nblk, _B), jnp.int32),
            pltpu.VMEM((nblk, _B), jnp.int32),
            pltpu.VMEM((_K, _B, _DH), jnp.float32),
            pltpu.SemaphoreType.DMA((_K,)),
            pltpu.SemaphoreType.DMA((_K,)),
            pltpu.VMEM_SHARED((_NP, _DH), jnp.float32),
        ],
        compiler_params=pltpu.CompilerParams(use_tc_tiling_on_sc=False),
    )
    def agg_kernel(v_hbm, src_hbm, dst_hbm, out_hbm, srcv, dstv, rows,
                   gsem, ssem, acc):
        c = lax.axis_index("c")
        s = lax.axis_index("s")
        pltpu.sync_copy(src_hbm.at[pl.ds(c * _EBLK + s * nblk, nblk)], srcv)
        pltpu.sync_copy(dst_hbm.at[pl.ds(s * nblk, nblk)], dstv)
        pltpu.sync_copy(v_hbm.at[pl.ds(c * _NP + s * _RPS, _RPS)],
                        acc.at[pl.ds(s * _RPS, _RPS)])
        plsc.subcore_barrier()

        for b in range(_K):
            pltpu.async_copy(v_hbm.at[srcv.at[b]], rows.at[b], gsem.at[b])

        @pl.loop(0, nblk, step=_K)
        def _(j):
            for b in range(_K):
                pltpu.make_async_copy(v_hbm.at[srcv.at[0]], rows.at[b],
                                      gsem.at[b]).wait()
                pltpu.async_copy(rows.at[b], acc.at[dstv.at[j + b]],
                                 ssem.at[b], add=True)
            for b in range(_K):
                pltpu.make_async_copy(rows.at[b], acc.at[dstv.at[0]],
                                      ssem.at[b]).wait()

                @pl.when(j + _K < nblk)
                def _():
                    pltpu.async_copy(v_hbm.at[srcv.at[j + _K + b]],
                                     rows.at[b], gsem.at[b])

        plsc.subcore_barrier()
        pltpu.sync_copy(acc.at[pl.ds(s * _RPS, _RPS)],
                        out_hbm.at[pl.ds(c * _NP + s * _RPS, _RPS)])

    return agg_kernel


_deg = _make_deg_kernel()
_agg = _make_agg_kernel()


def _scale0_body(x_ref, cnt_ref, xs_ref, d_ref):
    c0 = cnt_ref[0:_N, 0:1]
    c1 = cnt_ref[_NP:_NP + _N, 0:1]
    d = lax.rsqrt(1.0 + c0 + c1)
    d_ref[...] = d
    xs = x_ref[...] * d
    xs_ref[0:_N, :] = xs[:, 0:64]
    xs_ref[_NP:_NP + _N, :] = xs[:, 64:128]


def _scale0(x, cnt):
    return pl.pallas_call(
        _scale0_body,
        out_shape=(jax.ShapeDtypeStruct((2 * _NP, _DH), jnp.float32),
                   jax.ShapeDtypeStruct((_N, 1), jnp.float32)),
    )(x, cnt)


def _mid_body(a_ref, d_ref, b1_ref, w1_ref, w2_ref, o_ref):
    d = d_ref[...]
    aggx = jnp.concatenate([a_ref[0:_N, :], a_ref[_NP:_NP + _N, :]], axis=1)
    h1 = jnp.maximum(
        jnp.dot(aggx * d, w1_ref[...], preferred_element_type=jnp.float32)
        + b1_ref[...], 0.0)
    g = jnp.dot(h1, w2_ref[...], preferred_element_type=jnp.float32) * d
    o_ref[0:_N, :] = g[:, 0:64]
    o_ref[_NP:_NP + _N, :] = g[:, 64:128]


def _mid(a1, d, b1, w1, w2):
    return pl.pallas_call(
        _mid_body,
        out_shape=jax.ShapeDtypeStruct((2 * _NP, _DH), jnp.float32),
    )(a1, d, b1, w1, w2)


def _fin_body(a_ref, d_ref, b2_ref, o_ref):
    d = d_ref[...]
    lo = a_ref[0:_N, :]
    hi = a_ref[_NP:_NP + _N, :]
    o_ref[...] = jnp.concatenate([lo, hi], axis=1) * d + b2_ref[...]


def _fin(acc2, d, b2):
    return pl.pallas_call(
        _fin_body,
        out_shape=jax.ShapeDtypeStruct((_N, 128), jnp.float32),
    )(acc2, d, b2)


def kernel(x, edge_index, W1, b1, W2, b2):
    src = edge_index[0]
    dst = edge_index[1]
    npad = _EP - _E
    srcp = jnp.concatenate([src, jnp.zeros((npad,), jnp.int32)])
    dstp = jnp.concatenate([dst, jnp.full((npad,), _N, jnp.int32)])
    dst2 = dstp.reshape(_EBLK, _B)
    src2 = jnp.concatenate([srcp, srcp + _NP]).reshape(2 * _EBLK, _B)
    ones16 = jnp.ones((_B, 16), jnp.float32)
    zeros16 = jnp.zeros((_NP, 16), jnp.float32)

    cnt = _deg(dst2, ones16, zeros16)
    xs, d = _scale0(x, cnt)
    a1 = _agg(xs, src2, dst2)
    g = _mid(a1, d, b1.reshape(1, -1), W1, W2)
    a2 = _agg(g, src2, dst2)
    return _fin(a2, d, b2.reshape(1, -1))
